# Initial kernel scaffold; baseline (speedup 1.0000x reference)
import jax
import jax.numpy as jnp
from jax.experimental import pallas as pl


def kernel(x, edge_index, W_l0, W_r0, b0, W_l1, W_r1, b1):
    raise NotImplementedError("write your pallas kernel here")


# speedup vs baseline: 1.0891x; 1.0891x over previous
import dataclasses
import functools

import jax
import jax.numpy as jnp
from jax import lax
from jax.experimental import pallas as pl
from jax.experimental.pallas import tpu as pltpu
from jax.experimental.pallas import tpu_sc as plsc

N = 10000
E = 320000
D = 128
NS = 16
TR = 3456
NPASS = 3
CH = 128
NCHUNK = 160
EPAD = NS * NCHUNK * CH
L = 16

TRASH = TR
CNT_BASE = TR + 8
CROWS = 48
OUT_ROWS = 3584
ROWS_PT = OUT_ROWS // NS
ZROWS = 128

_mesh = plsc.VectorSubcoreMesh(core_axis_name="c", subcore_axis_name="s",
                               num_cores=1)

_scratch = [
    pltpu.VMEM((NCHUNK, CH), jnp.int32),
    pltpu.VMEM((NCHUNK, CH), jnp.int32),
    pltpu.VMEM((L,), jnp.int32),
    pltpu.VMEM((CH,), jnp.int32),
    pltpu.VMEM((CH,), jnp.int32),
    pltpu.VMEM((CH, D), jnp.float32),
    pltpu.VMEM((CH, D), jnp.float32),
    pltpu.VMEM((ZROWS, D), jnp.float32),
    pltpu.VMEM((CROWS, D), jnp.float32),
    pltpu.VMEM((CROWS,), jnp.int32),
    pltpu.VMEM_SHARED((OUT_ROWS, D), jnp.float32),
    pltpu.SemaphoreType.DMA,
    pltpu.SemaphoreType.DMA,
]

_cp = pltpu.CompilerParams()
if "needs_layout_passes" in pltpu.CompilerParams.__dataclass_fields__:
  _cp = dataclasses.replace(_cp, needs_layout_passes=False)


@functools.partial(
    pl.kernel,
    out_type=jax.ShapeDtypeStruct((OUT_ROWS, D), jnp.float32),
    mesh=_mesh,
    scratch_types=_scratch,
    compiler_params=_cp,
)
def _scatter_pass(table_hbm, src_r_hbm, dst_r_hbm, base_hbm, out_hbm,
                  src_v, dst_v, base_v, adj_a, adj_b, buf_a, buf_b, zbuf,
                  cnt_v, cidx_v, acc_sh, sem_a, sem_b):
  s = lax.axis_index("s")

  pltpu.sync_copy(base_hbm, base_v)
  zeros16 = jnp.zeros((L,), jnp.float32)

  @pl.loop(0, ZROWS)
  def _zero_rows(r):
    @pl.loop(0, D, step=L)
    def _zero_cols(col):
      zbuf[r, pl.ds(col, L)] = zeros16

  @pl.loop(0, CROWS)
  def _zero_cnt(r):
    @pl.loop(0, D, step=L)
    def _zero_ccols(col):
      cnt_v[r, pl.ds(col, L)] = zeros16

  @pl.loop(0, CROWS, step=L)
  def _fill_cidx(r):
    cidx_v[pl.ds(r, L)] = lax.iota(jnp.int32, L) + (CNT_BASE + r)

  @pl.loop(0, (ROWS_PT // ZROWS) * ZROWS, step=ZROWS)
  def _clear(r0):
    pltpu.sync_copy(zbuf, acc_sh.at[pl.ds(s * ROWS_PT + r0, ZROWS)])
  if ROWS_PT % ZROWS:
    pltpu.sync_copy(zbuf.at[pl.ds(0, ROWS_PT % ZROWS)],
                    acc_sh.at[pl.ds(s * ROWS_PT + (ROWS_PT // ZROWS) * ZROWS,
                                    ROWS_PT % ZROWS)])

  pltpu.sync_copy(src_r_hbm.at[s], src_v)
  pltpu.sync_copy(dst_r_hbm.at[s], dst_v)
  plsc.subcore_barrier()

  def localize(j, adj_v):
    @pl.loop(0, CH, step=L)
    def _adj(k):
      v = dst_v[j, pl.ds(k, L)]
      lv = v - base_v[...]
      valid = jnp.logical_and(lv >= 0, lv < TR)
      adj_v[pl.ds(k, L)] = jnp.where(valid, lv, TRASH)
      cnt, last = plsc.scan_count(v)
      plsc.addupdate_scatter(
          cnt_v,
          [lax.shift_right_logical(lv, 7), lax.bitwise_and(lv, 127)],
          cnt.astype(jnp.float32),
          mask=jnp.logical_and(last, valid),
      )

  pltpu.async_copy(table_hbm.at[src_v.at[0]], buf_a, sem_a)

  @pl.loop(0, NCHUNK, step=2)
  def _chunks(j):
    pltpu.async_copy(table_hbm.at[src_v.at[j + 1]], buf_b, sem_b)
    localize(j, adj_a)
    pltpu.make_async_copy(table_hbm.at[src_v.at[j]], buf_a, sem_a).wait()
    pltpu.sync_copy(buf_a, acc_sh.at[adj_a], add=True)

    @pl.when(j + 2 < NCHUNK)
    def _prefetch():
      pltpu.async_copy(table_hbm.at[src_v.at[j + 2]], buf_a, sem_a)

    localize(j + 1, adj_b)
    pltpu.make_async_copy(table_hbm.at[src_v.at[j + 1]], buf_b, sem_b).wait()
    pltpu.sync_copy(buf_b, acc_sh.at[adj_b], add=True)

  pltpu.sync_copy(cnt_v, acc_sh.at[cidx_v], add=True)
  plsc.subcore_barrier()

  @pl.loop(0, (ROWS_PT // ZROWS) * ZROWS, step=ZROWS)
  def _writeback(r0):
    pltpu.sync_copy(acc_sh.at[pl.ds(s * ROWS_PT + r0, ZROWS)],
                    out_hbm.at[pl.ds(s * ROWS_PT + r0, ZROWS)])
  if ROWS_PT % ZROWS:
    rem0 = (ROWS_PT // ZROWS) * ZROWS
    pltpu.sync_copy(acc_sh.at[pl.ds(s * ROWS_PT + rem0, ROWS_PT % ZROWS)],
                    out_hbm.at[pl.ds(s * ROWS_PT + rem0, ROWS_PT % ZROWS)])


NCROW = TR // D


def _tc_body(a0_ref, a1_ref, a2_ref, h_ref, wl_ref, wr_ref, b_ref, flag_ref,
             o_ref):
  i = pl.program_id(0)
  a = jnp.where(i == 0, a0_ref[...],
                jnp.where(i == 1, a1_ref[...], a2_ref[...]))
  eye = jnp.eye(D, dtype=jnp.float32)
  cols = [
      jnp.sum(jnp.broadcast_to(a[CNT_BASE + g:CNT_BASE + g + 1, :], (D, D))
              * eye, axis=1, keepdims=True)
      for g in range(NCROW)
  ]
  cnt = jnp.maximum(jnp.concatenate(cols, axis=0), 1.0)
  mean = a[:TR, :] / cnt
  o = (jnp.dot(mean, wl_ref[...], preferred_element_type=jnp.float32)
       + jnp.dot(h_ref[...], wr_ref[...], preferred_element_type=jnp.float32)
       + b_ref[...])
  o_ref[...] = jnp.where(flag_ref[0, 0] > 0, jnp.maximum(o, 0.0), o)


_tc_layer = pl.pallas_call(
    _tc_body,
    grid=(NPASS,),
    in_specs=[
        pl.BlockSpec((OUT_ROWS, D), lambda i: (0, 0)),
        pl.BlockSpec((OUT_ROWS, D), lambda i: (0, 0)),
        pl.BlockSpec((OUT_ROWS, D), lambda i: (0, 0)),
        pl.BlockSpec((TR, D), lambda i: (i, 0)),
        pl.BlockSpec((D, D), lambda i: (0, 0)),
        pl.BlockSpec((D, D), lambda i: (0, 0)),
        pl.BlockSpec((1, D), lambda i: (0, 0)),
        pl.BlockSpec((1, 1), lambda i: (0, 0)),
    ],
    out_specs=pl.BlockSpec((TR, D), lambda i: (i, 0)),
    out_shape=jax.ShapeDtypeStruct((N, D), jnp.float32),
)


def _pad_edges(ei):
  def body(i_ref, o_ref):
    o_ref[:, :E] = i_ref[...]
    o_ref[0:1, E:] = jnp.zeros((1, EPAD - E), jnp.int32)
    o_ref[1:2, E:] = jnp.full((1, EPAD - E), -1, jnp.int32)

  return pl.pallas_call(
      body,
      out_shape=jax.ShapeDtypeStruct((2, EPAD), jnp.int32),
  )(ei)


def kernel(x, edge_index, W_l0, W_r0, b0, W_l1, W_r1, b1):
  ei = _pad_edges(edge_index.astype(jnp.int32))
  src_r = ei[0].reshape(NS, NCHUNK, CH)
  dst_r = ei[1].reshape(NS, NCHUNK, CH)
  bases = [jnp.full((L,), t * TR, jnp.int32) for t in range(NPASS)]

  def layer(h, wl, wr, b, flag):
    accs = [_scatter_pass(h, src_r, dst_r, bases[t]) for t in range(NPASS)]
    return _tc_layer(accs[0], accs[1], accs[2], h, wl, wr,
                     b.reshape(1, D), jnp.float32(flag).reshape(1, 1))

  h0 = layer(x, W_l0, W_r0, b0, 1.0)
  return layer(h0, W_l1, W_r1, b1, -1.0)


# speedup vs baseline: 1.4311x; 1.3141x over previous
import dataclasses
import functools

import jax
import jax.numpy as jnp
from jax import lax
from jax.experimental import pallas as pl
from jax.experimental.pallas import tpu as pltpu
from jax.experimental.pallas import tpu_sc as plsc

N = 10000
E = 320000
D = 128
NS = 16
NT = 32
TR = 3456
NPASS = 3
CH = 128
NCHUNK = 80
EPAD = NT * NCHUNK * CH
L = 16

TRASH = TR
CNT_BASE = TR + 8
CROWS = 48
OUT_ROWS = 3584
ROWS_PT = OUT_ROWS // NS
ZROWS = 64

_mesh = plsc.VectorSubcoreMesh(core_axis_name="c", subcore_axis_name="s",
                               num_cores=2)

NBUF = 2

_scratch = (
    [
        pltpu.VMEM((NCHUNK, CH), jnp.int32),
        pltpu.VMEM((NCHUNK, CH), jnp.int32),
        pltpu.VMEM((L,), jnp.int32),
    ]
    + [pltpu.VMEM((CH,), jnp.int32) for _ in range(NBUF)]
    + [pltpu.VMEM((CH, D), jnp.float32) for _ in range(NBUF)]
    + [
        pltpu.VMEM((ZROWS, D), jnp.float32),
        pltpu.VMEM((CROWS, D), jnp.float32),
        pltpu.VMEM((CROWS,), jnp.int32),
        pltpu.VMEM_SHARED((OUT_ROWS, D), jnp.float32),
    ]
    + [pltpu.SemaphoreType.DMA for _ in range(2 * NBUF)]
)

_cp = pltpu.CompilerParams()
if "needs_layout_passes" in pltpu.CompilerParams.__dataclass_fields__:
  _cp = dataclasses.replace(_cp, needs_layout_passes=False)


@functools.partial(
    pl.kernel,
    out_type=jax.ShapeDtypeStruct((2, OUT_ROWS, D), jnp.float32),
    mesh=_mesh,
    scratch_types=_scratch,
    compiler_params=_cp,
)
def _scatter_pass(table_hbm, src_r_hbm, dst_r_hbm, base_hbm, out_hbm,
                  src_v, dst_v, base_v,
                  adj_0, adj_1,
                  buf_0, buf_1,
                  zbuf, cnt_v, cidx_v, acc_sh,
                  sg_0, sg_1, ss_0, ss_1):
  c = lax.axis_index("c")
  s = lax.axis_index("s")
  t = c * NS + s

  pltpu.sync_copy(base_hbm, base_v)
  zeros16 = jnp.zeros((L,), jnp.float32)

  @pl.loop(0, ZROWS)
  def _zero_rows(r):
    @pl.loop(0, D, step=L)
    def _zero_cols(col):
      zbuf[r, pl.ds(col, L)] = zeros16

  @pl.loop(0, CROWS)
  def _zero_cnt(r):
    @pl.loop(0, D, step=L)
    def _zero_ccols(col):
      cnt_v[r, pl.ds(col, L)] = zeros16

  @pl.loop(0, CROWS, step=L)
  def _fill_cidx(r):
    cidx_v[pl.ds(r, L)] = lax.iota(jnp.int32, L) + (CNT_BASE + r)

  @pl.loop(0, (ROWS_PT // ZROWS) * ZROWS, step=ZROWS)
  def _clear(r0):
    pltpu.sync_copy(zbuf, acc_sh.at[pl.ds(s * ROWS_PT + r0, ZROWS)])
  if ROWS_PT % ZROWS:
    pltpu.sync_copy(zbuf.at[pl.ds(0, ROWS_PT % ZROWS)],
                    acc_sh.at[pl.ds(s * ROWS_PT + (ROWS_PT // ZROWS) * ZROWS,
                                    ROWS_PT % ZROWS)])

  pltpu.sync_copy(src_r_hbm.at[t], src_v)
  pltpu.sync_copy(dst_r_hbm.at[t], dst_v)
  plsc.subcore_barrier()

  def localize(j, adj_v):
    @pl.loop(0, CH, step=L)
    def _adj(k):
      v = dst_v[j, pl.ds(k, L)]
      lv = v - base_v[...]
      valid = jnp.logical_and(lv >= 0, lv < TR)
      adj_v[pl.ds(k, L)] = jnp.where(valid, lv, TRASH)
      cnt, last = plsc.scan_count(v)
      plsc.addupdate_scatter(
          cnt_v,
          [lax.shift_right_logical(lv, 7), lax.bitwise_and(lv, 127)],
          cnt.astype(jnp.float32),
          mask=jnp.logical_and(last, valid),
      )

  bufs = (buf_0, buf_1)
  adjs = (adj_0, adj_1)
  sgs = (sg_0, sg_1)
  sss = (ss_0, ss_1)

  for n in range(NBUF):
    pltpu.async_copy(table_hbm.at[src_v.at[n]], bufs[n], sgs[n])

  @pl.loop(0, NCHUNK, step=NBUF)
  def _chunks(j):
    for n in range(NBUF):
      localize(j + n, adjs[n])
      pltpu.make_async_copy(table_hbm.at[src_v.at[j + n]], bufs[n],
                            sgs[n]).wait()
      pltpu.async_copy(bufs[n], acc_sh.at[adjs[n]], sss[n], add=True)

    @pl.when(j + NBUF < NCHUNK)
    def _prefetch():
      for n in range(NBUF):
        pltpu.make_async_copy(bufs[n], acc_sh.at[adjs[n]], sss[n]).wait()
        pltpu.async_copy(table_hbm.at[src_v.at[j + NBUF + n]], bufs[n],
                         sgs[n])

  for n in range(NBUF):
    pltpu.make_async_copy(bufs[n], acc_sh.at[adjs[n]], sss[n]).wait()

  pltpu.sync_copy(cnt_v, acc_sh.at[cidx_v], add=True)
  plsc.subcore_barrier()

  @pl.loop(0, (ROWS_PT // ZROWS) * ZROWS, step=ZROWS)
  def _writeback(r0):
    pltpu.sync_copy(acc_sh.at[pl.ds(s * ROWS_PT + r0, ZROWS)],
                    out_hbm.at[c, pl.ds(s * ROWS_PT + r0, ZROWS)])
  if ROWS_PT % ZROWS:
    rem0 = (ROWS_PT // ZROWS) * ZROWS
    pltpu.sync_copy(acc_sh.at[pl.ds(s * ROWS_PT + rem0, ROWS_PT % ZROWS)],
                    out_hbm.at[c, pl.ds(s * ROWS_PT + rem0, ROWS_PT % ZROWS)])


NCROW = TR // D


def _tc_body(a0_ref, a1_ref, a2_ref, h_ref, wl_ref, wr_ref, b_ref, flag_ref,
             o_ref):
  i = pl.program_id(0)
  a = jnp.where(i == 0, a0_ref[0] + a0_ref[1],
                jnp.where(i == 1, a1_ref[0] + a1_ref[1],
                          a2_ref[0] + a2_ref[1]))
  eye = jnp.eye(D, dtype=jnp.float32)
  cols = [
      jnp.sum(jnp.broadcast_to(a[CNT_BASE + g:CNT_BASE + g + 1, :], (D, D))
              * eye, axis=1, keepdims=True)
      for g in range(NCROW)
  ]
  cnt = jnp.maximum(jnp.concatenate(cols, axis=0), 1.0)
  mean = a[:TR, :] / cnt
  o = (jnp.dot(mean, wl_ref[...], preferred_element_type=jnp.float32)
       + jnp.dot(h_ref[...], wr_ref[...], preferred_element_type=jnp.float32)
       + b_ref[...])
  o_ref[...] = jnp.where(flag_ref[0, 0] > 0, jnp.maximum(o, 0.0), o)


_tc_layer = pl.pallas_call(
    _tc_body,
    grid=(NPASS,),
    in_specs=[
        pl.BlockSpec((2, OUT_ROWS, D), lambda i: (0, 0, 0)),
        pl.BlockSpec((2, OUT_ROWS, D), lambda i: (0, 0, 0)),
        pl.BlockSpec((2, OUT_ROWS, D), lambda i: (0, 0, 0)),
        pl.BlockSpec((TR, D), lambda i: (i, 0)),
        pl.BlockSpec((D, D), lambda i: (0, 0)),
        pl.BlockSpec((D, D), lambda i: (0, 0)),
        pl.BlockSpec((1, D), lambda i: (0, 0)),
        pl.BlockSpec((1, 1), lambda i: (0, 0)),
    ],
    out_specs=pl.BlockSpec((TR, D), lambda i: (i, 0)),
    out_shape=jax.ShapeDtypeStruct((N, D), jnp.float32),
)


def _pad_edges(ei):
  def body(i_ref, o_ref):
    o_ref[:, :E] = i_ref[...]
    o_ref[0:1, E:] = jnp.zeros((1, EPAD - E), jnp.int32)
    o_ref[1:2, E:] = jnp.full((1, EPAD - E), -1, jnp.int32)

  return pl.pallas_call(
      body,
      out_shape=jax.ShapeDtypeStruct((2, EPAD), jnp.int32),
  )(ei)


def kernel(x, edge_index, W_l0, W_r0, b0, W_l1, W_r1, b1):
  ei = _pad_edges(edge_index.astype(jnp.int32))
  src_r = ei[0].reshape(NT, NCHUNK, CH)
  dst_r = ei[1].reshape(NT, NCHUNK, CH)
  bases = [jnp.full((L,), t * TR, jnp.int32) for t in range(NPASS)]

  def layer(h, wl, wr, b, flag):
    accs = [_scatter_pass(h, src_r, dst_r, bases[t]) for t in range(NPASS)]
    return _tc_layer(accs[0], accs[1], accs[2], h, wl, wr,
                     b.reshape(1, D), jnp.float32(flag).reshape(1, 1))

  h0 = layer(x, W_l0, W_r0, b0, 1.0)
  return layer(h0, W_l1, W_r1, b1, -1.0)


# speedup vs baseline: 3.2714x; 2.2859x over previous
import dataclasses
import functools

import jax
import jax.numpy as jnp
from jax import lax
from jax.experimental import pallas as pl
from jax.experimental.pallas import tpu as pltpu
from jax.experimental.pallas import tpu_sc as plsc

N = 10000
E = 320000
D = 128
NS = 16
NT = 32
TR = 3456
NPASS = 3
CH = 128
NCHUNK = 80
EPAD = NT * NCHUNK * CH
PCAP = NCHUNK + 1
L = 16

TRASH = TR
TRASHPK = TRASH << 14
CNT_BASE = TR + 8
CROWS = 48
OUT_ROWS = 3584
ROWS_PT = OUT_ROWS // NS
ZROWS = 64

_mesh = plsc.VectorSubcoreMesh(core_axis_name="c", subcore_axis_name="s",
                               num_cores=2)

NBUF = 2

_cp = pltpu.CompilerParams()
if "needs_layout_passes" in pltpu.CompilerParams.__dataclass_fields__:
  _cp = dataclasses.replace(_cp, needs_layout_passes=False)


_part_scratch = (
    [
        pltpu.VMEM((NCHUNK, CH), jnp.int32),
        pltpu.VMEM((NCHUNK, CH), jnp.int32),
    ]
    + [pltpu.VMEM((PCAP, CH), jnp.int32) for _ in range(NPASS)]
    + [pltpu.VMEM((1, L), jnp.int32) for _ in range(NPASS)]
)


@functools.partial(
    pl.kernel,
    out_type=(
        jax.ShapeDtypeStruct((NT, NPASS, PCAP, CH), jnp.int32),
        jax.ShapeDtypeStruct((NT, NPASS, 1, L), jnp.int32),
    ),
    mesh=_mesh,
    scratch_types=_part_scratch,
    compiler_params=_cp,
)
def _partition(src_r_hbm, dst_r_hbm, pk_hbm, cnt_hbm,
               src_v, dst_v, bkt_0, bkt_1, bkt_2, off_0, off_1, off_2):
  c = lax.axis_index("c")
  s = lax.axis_index("s")
  t = c * NS + s
  bkts = (bkt_0, bkt_1, bkt_2)
  offs = (off_0, off_1, off_2)

  pltpu.sync_copy(src_r_hbm.at[t], src_v)
  pltpu.sync_copy(dst_r_hbm.at[t], dst_v)

  trash16 = jnp.full((L,), TRASHPK, jnp.int32)
  for b in range(NPASS):
    @pl.loop(0, PCAP)
    def _fillr(r, _b=b):
      @pl.loop(0, CH, step=L)
      def _fillc(cl):
        bkts[_b][r, pl.ds(cl, L)] = trash16
    offs[b][0, pl.ds(0, L)] = jnp.zeros((L,), jnp.int32)

  lane0 = lax.iota(jnp.int32, L) == 0

  @pl.loop(0, NCHUNK)
  def _chunks(j):
    @pl.loop(0, CH, step=L)
    def _groups(k):
      vs = src_v[j, pl.ds(k, L)]
      vd = dst_v[j, pl.ds(k, L)]
      for b in range(NPASS):
        lv = vd - b * TR
        m = jnp.logical_and(lv >= 0, lv < TR)
        mi = m.astype(jnp.int32)
        pk = jnp.bitwise_or(vs, lax.shift_left(lv, 14))
        rank = plsc.cumsum(mi) - 1
        base = jnp.sum(offs[b][0, pl.ds(0, L)])
        idx = rank + base
        plsc.store_scatter(
            bkts[b],
            [lax.shift_right_logical(idx, 7), lax.bitwise_and(idx, 127)],
            pk, mask=m)
        offs[b][0, pl.ds(0, L)] = (offs[b][0, pl.ds(0, L)]
                                   + jnp.where(lane0, jnp.sum(mi), 0))

  for b in range(NPASS):
    pltpu.sync_copy(offs[b], cnt_hbm.at[t, b])
    pltpu.sync_copy(bkts[b], pk_hbm.at[t, b])


_scat_scratch = (
    [
        pltpu.VMEM((PCAP, CH), jnp.int32),
        pltpu.VMEM((L,), jnp.int32),
        pltpu.VMEM((1, L), jnp.int32),
    ]
    + [pltpu.VMEM((CH,), jnp.int32) for _ in range(NBUF)]
    + [pltpu.VMEM((CH,), jnp.int32) for _ in range(NBUF)]
    + [pltpu.VMEM((CH, D), jnp.float32) for _ in range(NBUF)]
    + [
        pltpu.VMEM((ZROWS, D), jnp.float32),
        pltpu.VMEM((CROWS, D), jnp.float32),
        pltpu.VMEM((CROWS,), jnp.int32),
        pltpu.VMEM_SHARED((OUT_ROWS, D), jnp.float32),
    ]
    + [pltpu.SemaphoreType.DMA for _ in range(2 * NBUF)]
)


@functools.partial(
    pl.kernel,
    out_type=jax.ShapeDtypeStruct((2, OUT_ROWS, D), jnp.float32),
    mesh=_mesh,
    scratch_types=_scat_scratch,
    compiler_params=_cp,
)
def _scatter_pass(table_hbm, pk_hbm, cnt_hbm, sel_hbm, out_hbm,
                  pk_v, sel_v, cntv,
                  adj_0, adj_1, src_0, src_1, buf_0, buf_1,
                  zbuf, cnt_v, cidx_v, acc_sh,
                  sg_0, sg_1, ss_0, ss_1):
  c = lax.axis_index("c")
  s = lax.axis_index("s")
  t = c * NS + s

  pltpu.sync_copy(sel_hbm, sel_v)
  b3 = jnp.sum(sel_v[...])
  pltpu.sync_copy(cnt_hbm.at[t, b3], cntv)
  pltpu.sync_copy(pk_hbm.at[t, b3], pk_v)
  n_edges = jnp.sum(cntv[0, pl.ds(0, L)])

  zeros16 = jnp.zeros((L,), jnp.float32)

  @pl.loop(0, ZROWS)
  def _zero_rows(r):
    @pl.loop(0, D, step=L)
    def _zero_cols(col):
      zbuf[r, pl.ds(col, L)] = zeros16

  @pl.loop(0, CROWS)
  def _zero_cnt(r):
    @pl.loop(0, D, step=L)
    def _zero_ccols(col):
      cnt_v[r, pl.ds(col, L)] = zeros16

  @pl.loop(0, CROWS, step=L)
  def _fill_cidx(r):
    cidx_v[pl.ds(r, L)] = lax.iota(jnp.int32, L) + (CNT_BASE + r)

  @pl.loop(0, (ROWS_PT // ZROWS) * ZROWS, step=ZROWS)
  def _clear(r0):
    pltpu.sync_copy(zbuf, acc_sh.at[pl.ds(s * ROWS_PT + r0, ZROWS)])
  if ROWS_PT % ZROWS:
    pltpu.sync_copy(zbuf.at[pl.ds(0, ROWS_PT % ZROWS)],
                    acc_sh.at[pl.ds(s * ROWS_PT + (ROWS_PT // ZROWS) * ZROWS,
                                    ROWS_PT % ZROWS)])
  plsc.subcore_barrier()

  nch = lax.shift_right_logical(n_edges + (CH - 1), 7)
  nchr = jnp.maximum(
      lax.shift_left(lax.shift_right_logical(nch + (NBUF - 1), 1), 1), NBUF)

  def unpack(j, adj_v, src_b):
    @pl.loop(0, CH, step=L)
    def _un(k):
      p = pk_v[j, pl.ds(k, L)]
      lv = lax.shift_right_logical(p, 14)
      adj_v[pl.ds(k, L)] = lv
      src_b[pl.ds(k, L)] = jnp.bitwise_and(p, 16383)
      hc, last = plsc.scan_count(lv)
      plsc.addupdate_scatter(
          cnt_v,
          [lax.shift_right_logical(lv, 7), lax.bitwise_and(lv, 127)],
          hc.astype(jnp.float32),
          mask=jnp.logical_and(last, lv < TR),
      )

  bufs = (buf_0, buf_1)
  adjs = (adj_0, adj_1)
  srcs = (src_0, src_1)
  sgs = (sg_0, sg_1)
  sss = (ss_0, ss_1)

  for n in range(NBUF):
    unpack(n, adjs[n], srcs[n])
    pltpu.async_copy(table_hbm.at[srcs[n]], bufs[n], sgs[n])

  @pl.loop(0, nchr, step=NBUF)
  def _chunks(j):
    for n in range(NBUF):
      pltpu.make_async_copy(table_hbm.at[srcs[n]], bufs[n], sgs[n]).wait()
      pltpu.async_copy(bufs[n], acc_sh.at[adjs[n]], sss[n], add=True)

    @pl.when(j + NBUF < nchr)
    def _prefetch():
      for n in range(NBUF):
        pltpu.make_async_copy(bufs[n], acc_sh.at[adjs[n]], sss[n]).wait()
        unpack(j + NBUF + n, adjs[n], srcs[n])
        pltpu.async_copy(table_hbm.at[srcs[n]], bufs[n], sgs[n])

  for n in range(NBUF):
    pltpu.make_async_copy(bufs[n], acc_sh.at[adjs[n]], sss[n]).wait()

  pltpu.sync_copy(cnt_v, acc_sh.at[cidx_v], add=True)
  plsc.subcore_barrier()

  @pl.loop(0, (ROWS_PT // ZROWS) * ZROWS, step=ZROWS)
  def _writeback(r0):
    pltpu.sync_copy(acc_sh.at[pl.ds(s * ROWS_PT + r0, ZROWS)],
                    out_hbm.at[c, pl.ds(s * ROWS_PT + r0, ZROWS)])
  if ROWS_PT % ZROWS:
    rem0 = (ROWS_PT // ZROWS) * ZROWS
    pltpu.sync_copy(acc_sh.at[pl.ds(s * ROWS_PT + rem0, ROWS_PT % ZROWS)],
                    out_hbm.at[c, pl.ds(s * ROWS_PT + rem0, ROWS_PT % ZROWS)])


NCROW = TR // D


def _tc_body(a0_ref, a1_ref, a2_ref, h_ref, wl_ref, wr_ref, b_ref, flag_ref,
             o_ref):
  i = pl.program_id(0)
  a = jnp.where(i == 0, a0_ref[0] + a0_ref[1],
                jnp.where(i == 1, a1_ref[0] + a1_ref[1],
                          a2_ref[0] + a2_ref[1]))
  eye = jnp.eye(D, dtype=jnp.float32)
  cols = [
      jnp.sum(jnp.broadcast_to(a[CNT_BASE + g:CNT_BASE + g + 1, :], (D, D))
              * eye, axis=1, keepdims=True)
      for g in range(NCROW)
  ]
  cnt = jnp.maximum(jnp.concatenate(cols, axis=0), 1.0)
  mean = a[:TR, :] / cnt
  o = (jnp.dot(mean, wl_ref[...], preferred_element_type=jnp.float32)
       + jnp.dot(h_ref[...], wr_ref[...], preferred_element_type=jnp.float32)
       + b_ref[...])
  o_ref[...] = jnp.where(flag_ref[0, 0] > 0, jnp.maximum(o, 0.0), o)


_tc_layer = pl.pallas_call(
    _tc_body,
    grid=(NPASS,),
    in_specs=[
        pl.BlockSpec((2, OUT_ROWS, D), lambda i: (0, 0, 0)),
        pl.BlockSpec((2, OUT_ROWS, D), lambda i: (0, 0, 0)),
        pl.BlockSpec((2, OUT_ROWS, D), lambda i: (0, 0, 0)),
        pl.BlockSpec((TR, D), lambda i: (i, 0)),
        pl.BlockSpec((D, D), lambda i: (0, 0)),
        pl.BlockSpec((D, D), lambda i: (0, 0)),
        pl.BlockSpec((1, D), lambda i: (0, 0)),
        pl.BlockSpec((1, 1), lambda i: (0, 0)),
    ],
    out_specs=pl.BlockSpec((TR, D), lambda i: (i, 0)),
    out_shape=jax.ShapeDtypeStruct((N, D), jnp.float32),
)


def _pad_edges(ei):
  def body(i_ref, o_ref):
    o_ref[:, :E] = i_ref[...]
    o_ref[0:1, E:] = jnp.zeros((1, EPAD - E), jnp.int32)
    o_ref[1:2, E:] = jnp.full((1, EPAD - E), -1, jnp.int32)

  return pl.pallas_call(
      body,
      out_shape=jax.ShapeDtypeStruct((2, EPAD), jnp.int32),
  )(ei)


def kernel(x, edge_index, W_l0, W_r0, b0, W_l1, W_r1, b1):
  ei = _pad_edges(edge_index.astype(jnp.int32))
  src_r = ei[0].reshape(NT, NCHUNK, CH)
  dst_r = ei[1].reshape(NT, NCHUNK, CH)

  pk, cnts = _partition(src_r, dst_r)
  sels = [
      jnp.where(jnp.arange(L) == 0, b, 0).astype(jnp.int32)
      for b in range(NPASS)
  ]

  def layer(h, wl, wr, b, flag):
    accs = [_scatter_pass(h, pk, cnts, sels[t]) for t in range(NPASS)]
    return _tc_layer(accs[0], accs[1], accs[2], h, wl, wr,
                     b.reshape(1, D), jnp.float32(flag).reshape(1, 1))

  h0 = layer(x, W_l0, W_r0, b0, 1.0)
  return layer(h0, W_l1, W_r1, b1, -1.0)


# speedup vs baseline: 3.2934x; 1.0067x over previous
import dataclasses
import functools

import jax
import jax.numpy as jnp
from jax import lax
from jax.experimental import pallas as pl
from jax.experimental.pallas import tpu as pltpu
from jax.experimental.pallas import tpu_sc as plsc

N = 10000
E = 320000
D = 128
NS = 16
NT = 32
TR = 3456
NPASS = 3
CH = 128
NCHUNK = 80
EPAD = NT * NCHUNK * CH
PCAP = NCHUNK + 1
L = 16

TRASH = TR
TRASHPK = TRASH << 14
CNT_BASE = TR + 8
CROWS = 48
OUT_ROWS = 3584
ROWS_PT = OUT_ROWS // NS
ZROWS = 64

_mesh = plsc.VectorSubcoreMesh(core_axis_name="c", subcore_axis_name="s",
                               num_cores=2)

NBUF = 2

_cp = pltpu.CompilerParams()
if "needs_layout_passes" in pltpu.CompilerParams.__dataclass_fields__:
  _cp = dataclasses.replace(_cp, needs_layout_passes=False)


_part_scratch = (
    [
        pltpu.VMEM((NCHUNK, CH), jnp.int32),
        pltpu.VMEM((NCHUNK, CH), jnp.int32),
    ]
    + [pltpu.VMEM((PCAP, CH), jnp.int32) for _ in range(NPASS)]
    + [pltpu.VMEM((1, L), jnp.int32) for _ in range(NPASS)]
)


@functools.partial(
    pl.kernel,
    out_type=(
        jax.ShapeDtypeStruct((NT, NPASS, PCAP, CH), jnp.int32),
        jax.ShapeDtypeStruct((NT, NPASS, 1, L), jnp.int32),
    ),
    mesh=_mesh,
    scratch_types=_part_scratch,
    compiler_params=_cp,
)
def _partition(src_r_hbm, dst_r_hbm, pk_hbm, cnt_hbm,
               src_v, dst_v, bkt_0, bkt_1, bkt_2, off_0, off_1, off_2):
  c = lax.axis_index("c")
  s = lax.axis_index("s")
  t = c * NS + s
  bkts = (bkt_0, bkt_1, bkt_2)
  offs = (off_0, off_1, off_2)

  pltpu.sync_copy(src_r_hbm.at[t], src_v)
  pltpu.sync_copy(dst_r_hbm.at[t], dst_v)

  trash16 = jnp.full((L,), TRASHPK, jnp.int32)
  for b in range(NPASS):
    @pl.loop(0, PCAP)
    def _fillr(r, _b=b):
      @pl.loop(0, CH, step=L)
      def _fillc(cl):
        bkts[_b][r, pl.ds(cl, L)] = trash16
    offs[b][0, pl.ds(0, L)] = jnp.zeros((L,), jnp.int32)

  lane0 = lax.iota(jnp.int32, L) == 0

  @pl.loop(0, NCHUNK)
  def _chunks(j):
    @pl.loop(0, CH, step=L)
    def _groups(k):
      vs = src_v[j, pl.ds(k, L)]
      vd = dst_v[j, pl.ds(k, L)]
      for b in range(NPASS):
        lv = vd - b * TR
        m = jnp.logical_and(lv >= 0, lv < TR)
        mi = m.astype(jnp.int32)
        pk = jnp.bitwise_or(vs, lax.shift_left(lv, 14))
        rank = plsc.cumsum(mi) - 1
        base = jnp.sum(offs[b][0, pl.ds(0, L)])
        idx = rank + base
        plsc.store_scatter(
            bkts[b],
            [lax.shift_right_logical(idx, 7), lax.bitwise_and(idx, 127)],
            pk, mask=m)
        offs[b][0, pl.ds(0, L)] = (offs[b][0, pl.ds(0, L)]
                                   + jnp.where(lane0, jnp.sum(mi), 0))

  for b in range(NPASS):
    pltpu.sync_copy(offs[b], cnt_hbm.at[t, b])
    pltpu.sync_copy(bkts[b], pk_hbm.at[t, b])


_scat_scratch = (
    [
        pltpu.VMEM((PCAP, CH), jnp.int32),
        pltpu.VMEM((L,), jnp.int32),
        pltpu.VMEM((1, L), jnp.int32),
    ]
    + [pltpu.VMEM((CH,), jnp.int32) for _ in range(NBUF)]
    + [pltpu.VMEM((CH,), jnp.int32) for _ in range(NBUF)]
    + [pltpu.VMEM((CH, D), jnp.float32) for _ in range(NBUF)]
    + [
        pltpu.VMEM((ZROWS, D), jnp.float32),
        pltpu.VMEM((CROWS, D), jnp.float32),
        pltpu.VMEM((CROWS,), jnp.int32),
        pltpu.VMEM_SHARED((OUT_ROWS, D), jnp.float32),
    ]
    + [pltpu.SemaphoreType.DMA for _ in range(2 * NBUF + 1)]
)


@functools.partial(
    pl.kernel,
    out_type=jax.ShapeDtypeStruct((2, OUT_ROWS, D), jnp.float32),
    mesh=_mesh,
    scratch_types=_scat_scratch,
    compiler_params=_cp,
)
def _scatter_pass(table_hbm, pk_hbm, cnt_hbm, sel_hbm, out_hbm,
                  pk_v, sel_v, cntv,
                  adj_0, adj_1, src_0, src_1, buf_0, buf_1,
                  zbuf, cnt_v, cidx_v, acc_sh,
                  sg_0, sg_1, ss_0, ss_1, st_0):
  c = lax.axis_index("c")
  s = lax.axis_index("s")
  t = c * NS + s

  pltpu.sync_copy(sel_hbm, sel_v)
  b3 = jnp.sum(sel_v[...])
  pltpu.sync_copy(cnt_hbm.at[t, b3], cntv)
  pltpu.async_copy(pk_hbm.at[t, b3], pk_v, st_0)
  n_edges = jnp.sum(cntv[0, pl.ds(0, L)])

  zeros16 = jnp.zeros((L,), jnp.float32)

  @pl.loop(0, ZROWS)
  def _zero_rows(r):
    @pl.loop(0, D, step=L)
    def _zero_cols(col):
      zbuf[r, pl.ds(col, L)] = zeros16

  @pl.loop(0, CROWS)
  def _zero_cnt(r):
    @pl.loop(0, D, step=L)
    def _zero_ccols(col):
      cnt_v[r, pl.ds(col, L)] = zeros16

  @pl.loop(0, CROWS, step=L)
  def _fill_cidx(r):
    cidx_v[pl.ds(r, L)] = lax.iota(jnp.int32, L) + (CNT_BASE + r)

  @pl.loop(0, (ROWS_PT // ZROWS) * ZROWS, step=ZROWS)
  def _clear(r0):
    pltpu.sync_copy(zbuf, acc_sh.at[pl.ds(s * ROWS_PT + r0, ZROWS)])
  if ROWS_PT % ZROWS:
    pltpu.sync_copy(zbuf.at[pl.ds(0, ROWS_PT % ZROWS)],
                    acc_sh.at[pl.ds(s * ROWS_PT + (ROWS_PT // ZROWS) * ZROWS,
                                    ROWS_PT % ZROWS)])
  pltpu.make_async_copy(pk_hbm.at[t, b3], pk_v, st_0).wait()
  plsc.subcore_barrier()

  nch = lax.shift_right_logical(n_edges + (CH - 1), 7)
  nchr = jnp.maximum(
      lax.shift_left(lax.shift_right_logical(nch + (NBUF - 1), 1), 1), NBUF)

  def unpack(j, adj_v, src_b):
    @pl.loop(0, CH, step=L)
    def _un(k):
      p = pk_v[j, pl.ds(k, L)]
      lv = lax.shift_right_logical(p, 14)
      adj_v[pl.ds(k, L)] = lv
      src_b[pl.ds(k, L)] = jnp.bitwise_and(p, 16383)
      hc, last = plsc.scan_count(lv)
      plsc.addupdate_scatter(
          cnt_v,
          [lax.shift_right_logical(lv, 7), lax.bitwise_and(lv, 127)],
          hc.astype(jnp.float32),
          mask=jnp.logical_and(last, lv < TR),
      )

  bufs = (buf_0, buf_1)
  adjs = (adj_0, adj_1)
  srcs = (src_0, src_1)
  sgs = (sg_0, sg_1)
  sss = (ss_0, ss_1)

  for n in range(NBUF):
    unpack(n, adjs[n], srcs[n])
    pltpu.async_copy(table_hbm.at[srcs[n]], bufs[n], sgs[n])

  @pl.loop(0, nchr, step=NBUF)
  def _chunks(j):
    for n in range(NBUF):
      pltpu.make_async_copy(table_hbm.at[srcs[n]], bufs[n], sgs[n]).wait()
      pltpu.async_copy(bufs[n], acc_sh.at[adjs[n]], sss[n], add=True)

    @pl.when(j + NBUF < nchr)
    def _prefetch():
      for n in range(NBUF):
        pltpu.make_async_copy(bufs[n], acc_sh.at[adjs[n]], sss[n]).wait()
        unpack(j + NBUF + n, adjs[n], srcs[n])
        pltpu.async_copy(table_hbm.at[srcs[n]], bufs[n], sgs[n])

  for n in range(NBUF):
    pltpu.make_async_copy(bufs[n], acc_sh.at[adjs[n]], sss[n]).wait()

  pltpu.sync_copy(cnt_v, acc_sh.at[cidx_v], add=True)
  plsc.subcore_barrier()

  @pl.loop(0, (ROWS_PT // ZROWS) * ZROWS, step=ZROWS)
  def _writeback(r0):
    pltpu.sync_copy(acc_sh.at[pl.ds(s * ROWS_PT + r0, ZROWS)],
                    out_hbm.at[c, pl.ds(s * ROWS_PT + r0, ZROWS)])
  if ROWS_PT % ZROWS:
    rem0 = (ROWS_PT // ZROWS) * ZROWS
    pltpu.sync_copy(acc_sh.at[pl.ds(s * ROWS_PT + rem0, ROWS_PT % ZROWS)],
                    out_hbm.at[c, pl.ds(s * ROWS_PT + rem0, ROWS_PT % ZROWS)])


NCROW = TR // D


def _tc_body(a0_ref, a1_ref, a2_ref, h_ref, wl_ref, wr_ref, b_ref, flag_ref,
             o_ref):
  i = pl.program_id(0)
  a = jnp.where(i == 0, a0_ref[0] + a0_ref[1],
                jnp.where(i == 1, a1_ref[0] + a1_ref[1],
                          a2_ref[0] + a2_ref[1]))
  eye = jnp.eye(D, dtype=jnp.float32)
  cols = [
      jnp.sum(jnp.broadcast_to(a[CNT_BASE + g:CNT_BASE + g + 1, :], (D, D))
              * eye, axis=1, keepdims=True)
      for g in range(NCROW)
  ]
  cnt = jnp.maximum(jnp.concatenate(cols, axis=0), 1.0)
  mean = a[:TR, :] / cnt
  o = (jnp.dot(mean, wl_ref[...], preferred_element_type=jnp.float32)
       + jnp.dot(h_ref[...], wr_ref[...], preferred_element_type=jnp.float32)
       + b_ref[...])
  o_ref[...] = jnp.where(flag_ref[0, 0] > 0, jnp.maximum(o, 0.0), o)


_tc_layer = pl.pallas_call(
    _tc_body,
    grid=(NPASS,),
    in_specs=[
        pl.BlockSpec((2, OUT_ROWS, D), lambda i: (0, 0, 0)),
        pl.BlockSpec((2, OUT_ROWS, D), lambda i: (0, 0, 0)),
        pl.BlockSpec((2, OUT_ROWS, D), lambda i: (0, 0, 0)),
        pl.BlockSpec((TR, D), lambda i: (i, 0)),
        pl.BlockSpec((D, D), lambda i: (0, 0)),
        pl.BlockSpec((D, D), lambda i: (0, 0)),
        pl.BlockSpec((1, D), lambda i: (0, 0)),
        pl.BlockSpec((1, 1), lambda i: (0, 0)),
    ],
    out_specs=pl.BlockSpec((TR, D), lambda i: (i, 0)),
    out_shape=jax.ShapeDtypeStruct((N, D), jnp.float32),
)


def _pad_edges(ei):
  def body(i_ref, o_ref):
    o_ref[:, :E] = i_ref[...]
    o_ref[0:1, E:] = jnp.zeros((1, EPAD - E), jnp.int32)
    o_ref[1:2, E:] = jnp.full((1, EPAD - E), -1, jnp.int32)

  return pl.pallas_call(
      body,
      out_shape=jax.ShapeDtypeStruct((2, EPAD), jnp.int32),
  )(ei)


def kernel(x, edge_index, W_l0, W_r0, b0, W_l1, W_r1, b1):
  ei = _pad_edges(edge_index.astype(jnp.int32))
  src_r = ei[0].reshape(NT, NCHUNK, CH)
  dst_r = ei[1].reshape(NT, NCHUNK, CH)

  pk, cnts = _partition(src_r, dst_r)
  sels = [
      jnp.where(jnp.arange(L) == 0, b, 0).astype(jnp.int32)
      for b in range(NPASS)
  ]

  def layer(h, wl, wr, b, flag):
    accs = [_scatter_pass(h, pk, cnts, sels[t]) for t in range(NPASS)]
    return _tc_layer(accs[0], accs[1], accs[2], h, wl, wr,
                     b.reshape(1, D), jnp.float32(flag).reshape(1, 1))

  h0 = layer(x, W_l0, W_r0, b0, 1.0)
  return layer(h0, W_l1, W_r1, b1, -1.0)
